# baseline (speedup 1.0000x reference)
import functools

import jax
import jax.numpy as jnp
from jax import lax
from jax.experimental import pallas as pl
from jax.experimental.pallas import tpu as pltpu
from jax.experimental.pallas import tpu_sc as plsc

NC = 2
NS = 16
NW = NC * NS
CH = 128
G = 4


def _sc_scatter_call(d, ngroups, n_pad):
    rpz = n_pad // NS
    zfull = rpz // CH
    zrem = rpz % CH

    mesh = plsc.VectorSubcoreMesh(
        core_axis_name="c", subcore_axis_name="s", num_cores=NC,
        num_subcores=NS)

    @functools.partial(
        pl.kernel,
        mesh=mesh,
        out_type=jax.ShapeDtypeStruct((NC, n_pad, d), jnp.float32),
        scratch_types=[
            pltpu.VMEM((G, CH), jnp.int32),
            pltpu.VMEM((G, CH), jnp.int32),
            pltpu.VMEM((G, CH), jnp.int32),
            pltpu.VMEM((G, CH), jnp.int32),
            pltpu.VMEM((CH, d), jnp.float32),
            pltpu.VMEM((CH, d), jnp.float32),
            pltpu.VMEM_SHARED((n_pad, d), jnp.float32),
            pltpu.SemaphoreType.DMA,
            pltpu.SemaphoreType.DMA,
            pltpu.SemaphoreType.DMA,
            pltpu.SemaphoreType.DMA,
        ],
    )
    def scatter_kernel(rows_hbm, src_hbm, dst_hbm, out_hbm,
                       isrc0, isrc1, idst0, idst1, rbuf0, rbuf1, acc_sh,
                       gsem0, gsem1, isem0, isem1):
        c = lax.axis_index("c")
        s = lax.axis_index("s")
        wid = s * NC + c

        zero16 = jnp.zeros((16,), jnp.float32)

        def zrow(i, carry):
            for j in range(d // 16):
                rbuf0[i, pl.ds(j * 16, 16)] = zero16
            return carry

        lax.fori_loop(0, CH, zrow, 0)
        for k in range(zfull):
            pltpu.sync_copy(rbuf0, acc_sh.at[pl.ds(s * rpz + k * CH, CH)])
        if zrem:
            pltpu.sync_copy(
                rbuf0.at[pl.ds(0, zrem)],
                acc_sh.at[pl.ds(s * rpz + zfull * CH, zrem)])
        plsc.subcore_barrier()

        rbufs = (rbuf0, rbuf1)
        gsems = (gsem0, gsem1)

        def wait_rbuf(k):
            pltpu.make_async_copy(
                rows_hbm.at[isrc0.at[0]], rbufs[k % 2], gsems[k % 2]).wait()

        def process_group(isrc, idst, nsrc, nsrc_sems):
            for k in range(G):
                if k == G - 1:
                    for sem, buf in nsrc_sems:
                        pltpu.make_async_copy(
                            src_hbm.at[wid, 0], buf, sem).wait()
                    pltpu.async_copy(
                        rows_hbm.at[nsrc.at[0]],
                        rbufs[(k + 1) % 2], gsems[(k + 1) % 2])
                else:
                    pltpu.async_copy(
                        rows_hbm.at[isrc.at[k + 1]],
                        rbufs[(k + 1) % 2], gsems[(k + 1) % 2])
                wait_rbuf(k)
                pltpu.sync_copy(rbufs[k % 2], acc_sh.at[idst.at[k]], add=True)

        pltpu.sync_copy(src_hbm.at[wid, 0], isrc0)
        pltpu.sync_copy(dst_hbm.at[wid, 0], idst0)
        pltpu.async_copy(src_hbm.at[wid, 1], isrc1, isem1)
        pltpu.async_copy(dst_hbm.at[wid, 1], idst1, isem1)
        pltpu.async_copy(rows_hbm.at[isrc0.at[0]], rbuf0, gsem0)

        def body(i, carry):
            a = 2 * i
            process_group(isrc0, idst0, isrc1,
                          ((isem1, isrc1), (isem1, idst1)))
            pltpu.async_copy(src_hbm.at[wid, a + 2], isrc0, isem0)
            pltpu.async_copy(dst_hbm.at[wid, a + 2], idst0, isem0)
            process_group(isrc1, idst1, isrc0,
                          ((isem0, isrc0), (isem0, idst0)))
            pltpu.async_copy(src_hbm.at[wid, a + 3], isrc1, isem1)
            pltpu.async_copy(dst_hbm.at[wid, a + 3], idst1, isem1)
            return carry

        lax.fori_loop(0, ngroups // 2, body, 0)

        pltpu.make_async_copy(rows_hbm.at[isrc0.at[0]], rbuf0, gsem0).wait()
        pltpu.make_async_copy(src_hbm.at[wid, 0], isrc1, isem1).wait()
        pltpu.make_async_copy(dst_hbm.at[wid, 0], idst1, isem1).wait()
        plsc.subcore_barrier()

        pltpu.sync_copy(acc_sh.at[pl.ds(s * rpz, rpz)],
                        out_hbm.at[c, pl.ds(s * rpz, rpz)])

    return scatter_kernel


def _combine_matmul_relu_call(p, w, rows_blk):
    _, n, d = p.shape

    def body(p_ref, w_ref, o_ref):
        agg = p_ref[0] + p_ref[1]
        o_ref[...] = jnp.maximum(
            jnp.dot(agg, w_ref[...], preferred_element_type=jnp.float32), 0.0)

    return pl.pallas_call(
        body,
        grid=(n // rows_blk,),
        in_specs=[
            pl.BlockSpec((NC, rows_blk, d), lambda i: (0, i, 0)),
            pl.BlockSpec((d, d), lambda i: (0, 0)),
        ],
        out_specs=pl.BlockSpec((rows_blk, d), lambda i: (i, 0)),
        out_shape=jax.ShapeDtypeStruct((n, d), jnp.float32),
    )(p, w)


def kernel(A, X, W1, W2):
    x = X[0]
    n, d = x.shape
    e = A.shape[1]

    gsz = G * CH
    epw = -(-e // (NW * 2 * gsz)) * 2 * gsz
    ngroups = epw // gsz
    e_pad = NW * epw
    n_pad = -(-(n + 1) // 128) * 128

    trash = n + jnp.arange(e_pad - e, dtype=jnp.int32) % (n_pad - n)
    src = jnp.concatenate(
        [A[0], jnp.zeros((e_pad - e,), jnp.int32)]).reshape(
            NW, ngroups, G, CH)
    src = jnp.concatenate(
        [src, jnp.zeros((NW, 2, G, CH), jnp.int32)], axis=1)
    dst = jnp.concatenate([A[1], trash]).reshape(NW, ngroups, G, CH)
    dst = jnp.concatenate(
        [dst, jnp.full((NW, 2, G, CH), n, jnp.int32)], axis=1)

    scatter = _sc_scatter_call(d, ngroups, n_pad)

    blk = n_pad // 8
    p1 = scatter(x, src, dst)
    h1 = _combine_matmul_relu_call(p1, W1, blk)
    p2 = scatter(h1, src, dst)
    out = _combine_matmul_relu_call(p2, W2, blk)
    return out[None, :n, :]


# speedup vs baseline: 1.8323x; 1.8323x over previous
import functools

import jax
import jax.numpy as jnp
from jax import lax
from jax.experimental import pallas as pl
from jax.experimental.pallas import tpu as pltpu
from jax.experimental.pallas import tpu_sc as plsc

NC = 2
NS = 16
NW = NC * NS
CH = 128


def _sc_scatter_call(d, nchunk, n_pad):
    rpz = n_pad // NS
    zfull = rpz // CH
    zrem = rpz % CH

    mesh = plsc.VectorSubcoreMesh(
        core_axis_name="c", subcore_axis_name="s", num_cores=NC,
        num_subcores=NS)

    @functools.partial(
        pl.kernel,
        mesh=mesh,
        out_type=jax.ShapeDtypeStruct((NC, n_pad, d), jnp.float32),
        scratch_types=[
            pltpu.VMEM((nchunk, CH), jnp.int32),
            pltpu.VMEM((nchunk, CH), jnp.int32),
            pltpu.VMEM((CH, d), jnp.float32),
            pltpu.VMEM_SHARED((n_pad, d), jnp.float32),
            pltpu.SemaphoreType.DMA,
        ],
    )
    def scatter_kernel(rows_hbm, src_hbm, dst_hbm, out_hbm,
                       src_v, dst_v, rows_v, acc_sh, sem):
        c = lax.axis_index("c")
        s = lax.axis_index("s")
        wid = s * NC + c

        zero16 = jnp.zeros((16,), jnp.float32)

        def zrow(i, carry):
            for j in range(d // 16):
                rows_v[i, pl.ds(j * 16, 16)] = zero16
            return carry

        lax.fori_loop(0, CH, zrow, 0)
        for k in range(zfull):
            pltpu.sync_copy(rows_v, acc_sh.at[pl.ds(s * rpz + k * CH, CH)])
        if zrem:
            pltpu.sync_copy(
                rows_v.at[pl.ds(0, zrem)],
                acc_sh.at[pl.ds(s * rpz + zfull * CH, zrem)])
        plsc.subcore_barrier()

        pltpu.sync_copy(src_hbm.at[wid], src_v)
        pltpu.sync_copy(dst_hbm.at[wid], dst_v)

        def step(j, carry):
            pltpu.async_copy(rows_hbm.at[src_v.at[j]], rows_v, sem).wait()
            pltpu.sync_copy(rows_v, acc_sh.at[dst_v.at[j]], add=True)
            return carry

        lax.fori_loop(0, nchunk, step, 0)
        plsc.subcore_barrier()

        pltpu.sync_copy(acc_sh.at[pl.ds(s * rpz, rpz)],
                        out_hbm.at[c, pl.ds(s * rpz, rpz)])

    return scatter_kernel


def _combine_matmul_relu_call(p, w, rows_blk):
    _, n, d = p.shape

    def body(p_ref, w_ref, o_ref):
        agg = p_ref[0] + p_ref[1]
        o_ref[...] = jnp.maximum(
            jnp.dot(agg, w_ref[...], preferred_element_type=jnp.float32), 0.0)

    return pl.pallas_call(
        body,
        grid=(n // rows_blk,),
        in_specs=[
            pl.BlockSpec((NC, rows_blk, d), lambda i: (0, i, 0)),
            pl.BlockSpec((d, d), lambda i: (0, 0)),
        ],
        out_specs=pl.BlockSpec((rows_blk, d), lambda i: (i, 0)),
        out_shape=jax.ShapeDtypeStruct((n, d), jnp.float32),
    )(p, w)


def kernel(A, X, W1, W2):
    x = X[0]
    n, d = x.shape
    e = A.shape[1]

    epw = -(-e // (NW * CH)) * CH
    nchunk = epw // CH
    e_pad = NW * epw
    n_pad = -(-(n + 1) // 128) * 128

    trash = n + jnp.arange(e_pad - e, dtype=jnp.int32) % (n_pad - n)
    src = jnp.concatenate(
        [A[0], jnp.zeros((e_pad - e,), jnp.int32)]).reshape(NW, nchunk, CH)
    dst = jnp.concatenate([A[1], trash]).reshape(NW, nchunk, CH)

    scatter = _sc_scatter_call(d, nchunk, n_pad)

    blk = n_pad // 8
    p1 = scatter(x, src, dst)
    h1 = _combine_matmul_relu_call(p1, W1, blk)
    p2 = scatter(h1, src, dst)
    out = _combine_matmul_relu_call(p2, W2, blk)
    return out[None, :n, :]
